# baseline (speedup 1.0000x reference)
import functools

import jax
import jax.numpy as jnp
from jax import lax
from jax.experimental import pallas as pl
from jax.experimental.pallas import tpu as pltpu
from jax.experimental.pallas import tpu_sc as plsc

N = 10000
E = 320000
D_IN = 128
D = 64
D_OUT = 4

NC = 2
NS = 16
NW = NC * NS
K = 128
NBUF = 4
B = -(-(-(-E // (NW * K))) // NBUF) * NBUF
EPT = B * K
EP = NW * EPT
NROWS = 10240
DUMMY = N
ZROWS = NROWS // NS
DEGW = 16


def _sc_edge_body(g_hbm, src_hbm, dst_hbm, zeros_hbm, out_hbm,
                  acc, src_v, dst_v, rows_v, zbuf_v, gsem, ssem):
    c = lax.axis_index("c")
    s = lax.axis_index("s")
    wid = c * NS + s
    pltpu.sync_copy(zeros_hbm, zbuf_v)
    pltpu.sync_copy(src_hbm.at[wid], src_v)
    pltpu.sync_copy(dst_hbm.at[wid], dst_v)
    for k in range(ZROWS // K):
        pltpu.sync_copy(zbuf_v, acc.at[pl.ds(s * ZROWS + k * K, K)])
    plsc.subcore_barrier()

    def gather(j, b):
        pltpu.async_copy(g_hbm.at[src_v.at[j]], rows_v.at[b], gsem.at[b])

    def scatter(j, b):
        pltpu.async_copy(rows_v.at[b], acc.at[dst_v.at[j]], ssem.at[b],
                         add=True)

    def wait_g(b):
        pltpu.make_async_copy(g_hbm.at[src_v.at[0]], rows_v.at[b],
                              gsem.at[b]).wait()

    def wait_s(b):
        pltpu.make_async_copy(rows_v.at[b], acc.at[dst_v.at[0]],
                              ssem.at[b]).wait()

    for b in range(NBUF):
        gather(b, b)

    @pl.loop(0, B // NBUF - 1)
    def _(i):
        base = i * NBUF
        for b in range(NBUF):
            wait_g(b)
            scatter(base + b, b)
        for b in range(NBUF):
            wait_s(b)
            gather(base + NBUF + b, b)

    tail = B - NBUF
    for b in range(NBUF):
        wait_g(b)
        scatter(tail + b, b)
    for b in range(NBUF):
        wait_s(b)

    plsc.subcore_barrier()
    pltpu.sync_copy(
        acc.at[pl.ds(s * ZROWS, ZROWS)],
        out_hbm.at[c, pl.ds(s * ZROWS, ZROWS)],
    )


_DCHUNK = 8


def _sc_degree_body(dst_hbm, zeros_hbm, ones_hbm, out_hbm,
                    acc, dst_v, ones_v, zbuf_v, sem):
    c = lax.axis_index("c")
    s = lax.axis_index("s")
    wid = c * NS + s
    pltpu.sync_copy(zeros_hbm, zbuf_v)
    pltpu.sync_copy(ones_hbm, ones_v)
    pltpu.sync_copy(dst_hbm.at[wid], dst_v)
    for k in range(ZROWS // K):
        pltpu.sync_copy(zbuf_v, acc.at[pl.ds(s * ZROWS + k * K, K)])
    plsc.subcore_barrier()

    @pl.loop(0, B // _DCHUNK)
    def _(i):
        for k in range(_DCHUNK):
            pltpu.async_copy(ones_v, acc.at[dst_v.at[i * _DCHUNK + k]], sem,
                             add=True)
        for k in range(_DCHUNK):
            pltpu.make_async_copy(ones_v, acc.at[dst_v.at[0]], sem).wait()

    plsc.subcore_barrier()
    pltpu.sync_copy(
        acc.at[pl.ds(s * ZROWS, ZROWS)],
        out_hbm.at[c, pl.ds(s * ZROWS, ZROWS)],
    )


@functools.cache
def _sc_kernels():
    mesh = plsc.VectorSubcoreMesh(
        core_axis_name="c", subcore_axis_name="s", num_cores=NC, num_subcores=NS
    )
    params = pltpu.CompilerParams(use_tc_tiling_on_sc=False)
    edge = pl.kernel(
        _sc_edge_body,
        compiler_params=params,
        out_type=jax.ShapeDtypeStruct((NC, NROWS, D), jnp.float32),
        mesh=mesh,
        scratch_types=[
            pltpu.VMEM_SHARED((NROWS, D), jnp.float32),
            pltpu.VMEM((B, K), jnp.int32),
            pltpu.VMEM((B, K), jnp.int32),
            pltpu.VMEM((NBUF, K, D), jnp.float32),
            pltpu.VMEM((K, D), jnp.float32),
            pltpu.SemaphoreType.DMA((NBUF,)),
            pltpu.SemaphoreType.DMA((NBUF,)),
        ],
    )
    degree = pl.kernel(
        _sc_degree_body,
        compiler_params=params,
        out_type=jax.ShapeDtypeStruct((NC, NROWS, DEGW), jnp.float32),
        mesh=mesh,
        scratch_types=[
            pltpu.VMEM_SHARED((NROWS, DEGW), jnp.float32),
            pltpu.VMEM((B, K), jnp.int32),
            pltpu.VMEM((K, DEGW), jnp.float32),
            pltpu.VMEM((K, DEGW), jnp.float32),
            pltpu.SemaphoreType.DMA,
        ],
    )
    return edge, degree


R = 1000
GRID = N // R

_DOT = dict(preferred_element_type=jnp.float32, precision=lax.Precision.HIGHEST)


def _enc_body(x_ref, d0_ref, d1_ref, encW_ref, encb_ref, W1_ref,
              g1_ref, dinv_ref):
    deg = d0_ref[:, 0:1] + d1_ref[:, 0:1] + 1.0
    dinv = lax.rsqrt(deg)
    z = jnp.maximum(jnp.dot(x_ref[...], encW_ref[...], **_DOT) + encb_ref[...], 0.0)
    g1_ref[...] = jnp.dot(z, W1_ref[...], **_DOT) * dinv
    dinv_ref[...] = dinv


def _layer_body(s0_ref, s1_ref, g_ref, dinv_ref, b_ref, Wn_ref, gn_ref):
    dinv = dinv_ref[...]
    h = jax.nn.gelu(dinv * (s0_ref[...] + s1_ref[...] + g_ref[...]) + b_ref[...])
    gn_ref[...] = jnp.dot(h, Wn_ref[...], **_DOT) * dinv


def _final_body(s0_ref, s1_ref, g_ref, dinv_ref, b_ref, decW_ref, decb_ref,
                out_ref):
    dinv = dinv_ref[...]
    h = jax.nn.gelu(dinv * (s0_ref[...] + s1_ref[...] + g_ref[...]) + b_ref[...])
    out_ref[...] = jnp.dot(h, decW_ref[...], **_DOT) + decb_ref[...]


def _rows(shape):
    return pl.BlockSpec((R,) + shape[1:], lambda i: (i,) + (0,) * (len(shape) - 1))


def _whole(shape):
    return pl.BlockSpec(shape, lambda i: (0,) * len(shape))


def _tc_call(body, in_arrays, out_shape):
    in_specs = [_rows(a.shape) if a.shape[0] in (N, NROWS) else _whole(a.shape)
                for a in in_arrays]
    out_specs = jax.tree.map(lambda s: _rows(s.shape), out_shape)
    return pl.pallas_call(
        body,
        grid=(GRID,),
        in_specs=in_specs,
        out_specs=out_specs,
        out_shape=out_shape,
    )(*in_arrays)


def kernel(x, edge_index, enc_W, enc_b, W1, b1, W2, b2, W3, b3, W4, b4,
           dec_W, dec_b):
    pad = EP - E
    src = jnp.concatenate([edge_index[0], jnp.zeros((pad,), jnp.int32)])
    dst = jnp.concatenate([edge_index[1], jnp.full((pad,), DUMMY, jnp.int32)])
    srcR = src.reshape(NW, B, K)
    dstR = dst.reshape(NW, B, K)

    zeros_d = jnp.zeros((K, D), jnp.float32)
    zeros_w = jnp.zeros((K, DEGW), jnp.float32)
    ones_w = jnp.ones((K, DEGW), jnp.float32)

    edge_scatter, degree = _sc_kernels()
    deg_parts = degree(dstR, zeros_w, ones_w)

    f32 = jnp.float32
    g1, dinv = _tc_call(
        _enc_body,
        [x, deg_parts[0], deg_parts[1], enc_W, enc_b.reshape(1, D), W1],
        (jax.ShapeDtypeStruct((N, D), f32), jax.ShapeDtypeStruct((N, 1), f32)),
    )

    g = g1
    for b, Wn in ((b1, W2), (b2, W3), (b3, W4)):
        s_parts = edge_scatter(g, srcR, dstR, zeros_d)
        g = _tc_call(
            _layer_body,
            [s_parts[0], s_parts[1], g, dinv, b.reshape(1, D), Wn],
            jax.ShapeDtypeStruct((N, D), f32),
        )

    s_parts = edge_scatter(g, srcR, dstR, zeros_d)
    out = _tc_call(
        _final_body,
        [s_parts[0], s_parts[1], g, dinv, b4.reshape(1, D), dec_W,
         dec_b.reshape(1, D_OUT)],
        jax.ShapeDtypeStruct((N, D_OUT), f32),
    )
    return out


# speedup vs baseline: 1.0793x; 1.0793x over previous
import functools

import jax
import jax.numpy as jnp
from jax import lax
from jax.experimental import pallas as pl
from jax.experimental.pallas import tpu as pltpu
from jax.experimental.pallas import tpu_sc as plsc

N = 10000
E = 320000
D_IN = 128
D = 64
D_OUT = 4

NC = 2
NS = 16
NW = NC * NS
K = 128
NBUF = 4
B = -(-(-(-E // (NW * K))) // NBUF) * NBUF
EPT = B * K
EP = NW * EPT
NROWS = 10240
DUMMY = N
ZROWS = NROWS // NS
DEGW = 16


def _sc_edge_body(g_hbm, src_hbm, dst_hbm, zeros_hbm, out_hbm,
                  acc, src_v, dst_v, rows_v, zbuf_v, gsem, ssem):
    c = lax.axis_index("c")
    s = lax.axis_index("s")
    wid = c * NS + s
    pltpu.sync_copy(zeros_hbm, zbuf_v)
    pltpu.sync_copy(src_hbm.at[wid], src_v)
    pltpu.sync_copy(dst_hbm.at[wid], dst_v)
    for k in range(ZROWS // K):
        pltpu.sync_copy(zbuf_v, acc.at[pl.ds(s * ZROWS + k * K, K)])
    plsc.subcore_barrier()

    def gather(j, b):
        pltpu.async_copy(g_hbm.at[src_v.at[j]], rows_v.at[b], gsem.at[b])

    def scatter(j, b):
        pltpu.async_copy(rows_v.at[b], acc.at[dst_v.at[j]], ssem.at[b],
                         add=True)

    def wait_g(b):
        pltpu.make_async_copy(g_hbm.at[src_v.at[0]], rows_v.at[b],
                              gsem.at[b]).wait()

    def wait_s(b):
        pltpu.make_async_copy(rows_v.at[b], acc.at[dst_v.at[0]],
                              ssem.at[b]).wait()

    for b in range(NBUF):
        gather(b, b)

    @pl.loop(0, B // NBUF - 1)
    def _(i):
        base = i * NBUF
        for b in range(NBUF):
            wait_g(b)
            scatter(base + b, b)
        for b in range(NBUF):
            wait_s(b)
            gather(base + NBUF + b, b)

    tail = B - NBUF
    for b in range(NBUF):
        wait_g(b)
        scatter(tail + b, b)
    for b in range(NBUF):
        wait_s(b)

    plsc.subcore_barrier()
    pltpu.sync_copy(
        acc.at[pl.ds(s * ZROWS, ZROWS)],
        out_hbm.at[c, pl.ds(s * ZROWS, ZROWS)],
    )


_DCHUNK = 8


def _sc_degree_body(dst_hbm, zeros_hbm, ones_hbm, out_hbm,
                    acc, dst_v, ones_v, zbuf_v, sem):
    c = lax.axis_index("c")
    s = lax.axis_index("s")
    wid = c * NS + s
    pltpu.sync_copy(zeros_hbm, zbuf_v)
    pltpu.sync_copy(ones_hbm, ones_v)
    pltpu.sync_copy(dst_hbm.at[wid], dst_v)
    for k in range(ZROWS // K):
        pltpu.sync_copy(zbuf_v, acc.at[pl.ds(s * ZROWS + k * K, K)])
    plsc.subcore_barrier()

    @pl.loop(0, B // _DCHUNK)
    def _(i):
        for k in range(_DCHUNK):
            pltpu.async_copy(ones_v, acc.at[dst_v.at[i * _DCHUNK + k]], sem,
                             add=True)
        for k in range(_DCHUNK):
            pltpu.make_async_copy(ones_v, acc.at[dst_v.at[0]], sem).wait()

    plsc.subcore_barrier()
    pltpu.sync_copy(
        acc.at[pl.ds(s * ZROWS, ZROWS)],
        out_hbm.at[c, pl.ds(s * ZROWS, ZROWS)],
    )


@functools.cache
def _sc_kernels():
    mesh = plsc.VectorSubcoreMesh(
        core_axis_name="c", subcore_axis_name="s", num_cores=NC, num_subcores=NS
    )
    params = pltpu.CompilerParams(use_tc_tiling_on_sc=False)
    edge = pl.kernel(
        _sc_edge_body,
        compiler_params=params,
        out_type=jax.ShapeDtypeStruct((NC, NROWS, D), jnp.float32),
        mesh=mesh,
        scratch_types=[
            pltpu.VMEM_SHARED((NROWS, D), jnp.float32),
            pltpu.VMEM((B, K), jnp.int32),
            pltpu.VMEM((B, K), jnp.int32),
            pltpu.VMEM((NBUF, K, D), jnp.float32),
            pltpu.VMEM((K, D), jnp.float32),
            pltpu.SemaphoreType.DMA((NBUF,)),
            pltpu.SemaphoreType.DMA((NBUF,)),
        ],
    )
    degree = pl.kernel(
        _sc_degree_body,
        compiler_params=params,
        out_type=jax.ShapeDtypeStruct((NC, NROWS, DEGW), jnp.float32),
        mesh=mesh,
        scratch_types=[
            pltpu.VMEM_SHARED((NROWS, DEGW), jnp.float32),
            pltpu.VMEM((B, K), jnp.int32),
            pltpu.VMEM((K, DEGW), jnp.float32),
            pltpu.VMEM((K, DEGW), jnp.float32),
            pltpu.SemaphoreType.DMA,
        ],
    )
    return edge, degree


R = 1000
GRID = N // R

_DOT = dict(preferred_element_type=jnp.float32, precision=lax.Precision.HIGHEST)


def _enc_body(x_ref, d0_ref, d1_ref, encW_ref, encb_ref, W1_ref,
              g1_ref, dinv_ref):
    deg = d0_ref[:, 0:1] + d1_ref[:, 0:1] + 1.0
    dinv = lax.rsqrt(deg)
    z = jnp.maximum(jnp.dot(x_ref[...], encW_ref[...], **_DOT) + encb_ref[...], 0.0)
    g1_ref[...] = jnp.dot(z, W1_ref[...], **_DOT) * dinv
    dinv_ref[...] = dinv


def _layer_body(s0_ref, s1_ref, g_ref, dinv_ref, b_ref, Wn_ref, gn_ref):
    dinv = dinv_ref[...]
    h = jax.nn.gelu(dinv * (s0_ref[...] + s1_ref[...] + g_ref[...]) + b_ref[...])
    gn_ref[...] = jnp.dot(h, Wn_ref[...], **_DOT) * dinv


def _final_body(s0_ref, s1_ref, g_ref, dinv_ref, b_ref, decW_ref, decb_ref,
                out_ref):
    dinv = dinv_ref[...]
    h = jax.nn.gelu(dinv * (s0_ref[...] + s1_ref[...] + g_ref[...]) + b_ref[...])
    out_ref[...] = jnp.dot(h, decW_ref[...], **_DOT) + decb_ref[...]


def _rows(shape):
    return pl.BlockSpec((R,) + shape[1:], lambda i: (i,) + (0,) * (len(shape) - 1))


def _whole(shape):
    return pl.BlockSpec(shape, lambda i: (0,) * len(shape))


def _tc_call(body, in_arrays, out_shape):
    in_specs = [_rows(a.shape) if a.shape[0] in (N, NROWS) else _whole(a.shape)
                for a in in_arrays]
    out_specs = jax.tree.map(lambda s: _rows(s.shape), out_shape)
    return pl.pallas_call(
        body,
        grid=(GRID,),
        in_specs=in_specs,
        out_specs=out_specs,
        out_shape=out_shape,
    )(*in_arrays)


def kernel(x, edge_index, enc_W, enc_b, W1, b1, W2, b2, W3, b3, W4, b4,
           dec_W, dec_b):
    ept_real = E // NW
    padt = EPT - ept_real
    src_pad = jnp.zeros((NW, padt), jnp.int32)
    dst_pad = jnp.broadcast_to(
        DUMMY + (jnp.arange(padt, dtype=jnp.int32) % (NROWS - N)), (NW, padt))
    srcR = jnp.concatenate(
        [edge_index[0].reshape(NW, ept_real), src_pad], axis=1).reshape(NW, B, K)
    dstR = jnp.concatenate(
        [edge_index[1].reshape(NW, ept_real), dst_pad], axis=1).reshape(NW, B, K)

    zeros_d = jnp.zeros((K, D), jnp.float32)
    zeros_w = jnp.zeros((K, DEGW), jnp.float32)
    ones_w = jnp.ones((K, DEGW), jnp.float32)

    edge_scatter, degree = _sc_kernels()
    deg_parts = degree(dstR, zeros_w, ones_w)

    f32 = jnp.float32
    g1, dinv = _tc_call(
        _enc_body,
        [x, deg_parts[0], deg_parts[1], enc_W, enc_b.reshape(1, D), W1],
        (jax.ShapeDtypeStruct((N, D), f32), jax.ShapeDtypeStruct((N, 1), f32)),
    )

    g = g1
    for b, Wn in ((b1, W2), (b2, W3), (b3, W4)):
        s_parts = edge_scatter(g, srcR, dstR, zeros_d)
        g = _tc_call(
            _layer_body,
            [s_parts[0], s_parts[1], g, dinv, b.reshape(1, D), Wn],
            jax.ShapeDtypeStruct((N, D), f32),
        )

    s_parts = edge_scatter(g, srcR, dstR, zeros_d)
    out = _tc_call(
        _final_body,
        [s_parts[0], s_parts[1], g, dinv, b4.reshape(1, D), dec_W,
         dec_b.reshape(1, D_OUT)],
        jax.ShapeDtypeStruct((N, D_OUT), f32),
    )
    return out


# speedup vs baseline: 1.8924x; 1.7533x over previous
import functools

import jax
import jax.numpy as jnp
from jax import lax
from jax.experimental import pallas as pl
from jax.experimental.pallas import tpu as pltpu
from jax.experimental.pallas import tpu_sc as plsc

N = 10000
E = 320000
D_IN = 128
D = 64
D_OUT = 4

NC = 2
NS = 16
NW = NC * NS
K = 128
NBUF = 2
B = -(-(-(-E // (NW * K))) // NBUF) * NBUF
EPT = B * K
EP = NW * EPT
NROWS = 10240
DUMMY = N
ZROWS = NROWS // NS
DEGW = 16


def _sc_edge_body(g_hbm, src_hbm, dst_hbm, zeros_hbm, out_hbm,
                  acc, g_s, src_v, dst_v, rows_v, gsem, ssem):
    c = lax.axis_index("c")
    s = lax.axis_index("s")
    wid = c * NS + s
    pltpu.sync_copy(src_hbm.at[wid], src_v)
    pltpu.sync_copy(dst_hbm.at[wid], dst_v)
    last = N - (NS - 1) * ZROWS

    @pl.when(s < NS - 1)
    def _():
        pltpu.sync_copy(g_hbm.at[pl.ds(s * ZROWS, ZROWS)],
                        g_s.at[pl.ds(s * ZROWS, ZROWS)])

    @pl.when(s == NS - 1)
    def _():
        pltpu.sync_copy(g_hbm.at[pl.ds((NS - 1) * ZROWS, last)],
                        g_s.at[pl.ds((NS - 1) * ZROWS, last)])

    for k in range(ZROWS // K):
        pltpu.sync_copy(zeros_hbm, acc.at[pl.ds(s * ZROWS + k * K, K)])
    plsc.subcore_barrier()

    def gather(j, b):
        pltpu.async_copy(g_s.at[src_v.at[j]], rows_v.at[b], gsem.at[b])

    def scatter(j, b):
        pltpu.async_copy(rows_v.at[b], acc.at[dst_v.at[j]], ssem.at[b],
                         add=True)

    def wait_g(b):
        pltpu.make_async_copy(g_s.at[src_v.at[0]], rows_v.at[b],
                              gsem.at[b]).wait()

    def wait_s(b):
        pltpu.make_async_copy(rows_v.at[b], acc.at[dst_v.at[0]],
                              ssem.at[b]).wait()

    for b in range(NBUF):
        gather(b, b)

    @pl.loop(0, B // NBUF - 1)
    def _(i):
        base = i * NBUF
        for b in range(NBUF):
            wait_g(b)
            scatter(base + b, b)
        for b in range(NBUF):
            wait_s(b)
            gather(base + NBUF + b, b)

    tail = B - NBUF
    for b in range(NBUF):
        wait_g(b)
        scatter(tail + b, b)
    for b in range(NBUF):
        wait_s(b)

    plsc.subcore_barrier()
    pltpu.sync_copy(
        acc.at[pl.ds(s * ZROWS, ZROWS)],
        out_hbm.at[c, pl.ds(s * ZROWS, ZROWS)],
    )


_DCHUNK = 8


def _sc_degree_body(dst_hbm, zeros_hbm, ones_hbm, out_hbm,
                    acc, dst_v, ones_v, zbuf_v, sem):
    c = lax.axis_index("c")
    s = lax.axis_index("s")
    wid = c * NS + s
    pltpu.sync_copy(zeros_hbm, zbuf_v)
    pltpu.sync_copy(ones_hbm, ones_v)
    pltpu.sync_copy(dst_hbm.at[wid], dst_v)
    for k in range(ZROWS // K):
        pltpu.sync_copy(zbuf_v, acc.at[pl.ds(s * ZROWS + k * K, K)])
    plsc.subcore_barrier()

    @pl.loop(0, B // _DCHUNK)
    def _(i):
        for k in range(_DCHUNK):
            pltpu.async_copy(ones_v, acc.at[dst_v.at[i * _DCHUNK + k]], sem,
                             add=True)
        for k in range(_DCHUNK):
            pltpu.make_async_copy(ones_v, acc.at[dst_v.at[0]], sem).wait()

    plsc.subcore_barrier()
    pltpu.sync_copy(
        acc.at[pl.ds(s * ZROWS, ZROWS)],
        out_hbm.at[c, pl.ds(s * ZROWS, ZROWS)],
    )


@functools.cache
def _sc_kernels():
    mesh = plsc.VectorSubcoreMesh(
        core_axis_name="c", subcore_axis_name="s", num_cores=NC, num_subcores=NS
    )
    params = pltpu.CompilerParams(use_tc_tiling_on_sc=False)
    edge_params = pltpu.CompilerParams(use_tc_tiling_on_sc=False,
                                       internal_scratch_in_bytes=1 << 20)
    edge = pl.kernel(
        _sc_edge_body,
        compiler_params=edge_params,
        out_type=jax.ShapeDtypeStruct((NC, NROWS, D), jnp.float32),
        mesh=mesh,
        scratch_types=[
            pltpu.VMEM_SHARED((NROWS, D), jnp.float32),
            pltpu.VMEM_SHARED((NROWS, D), jnp.float32),
            pltpu.VMEM((B, K), jnp.int32),
            pltpu.VMEM((B, K), jnp.int32),
            pltpu.VMEM((NBUF, K, D), jnp.float32),
            pltpu.SemaphoreType.DMA((NBUF,)),
            pltpu.SemaphoreType.DMA((NBUF,)),
        ],
    )
    degree = pl.kernel(
        _sc_degree_body,
        compiler_params=params,
        out_type=jax.ShapeDtypeStruct((NC, NROWS, DEGW), jnp.float32),
        mesh=mesh,
        scratch_types=[
            pltpu.VMEM_SHARED((NROWS, DEGW), jnp.float32),
            pltpu.VMEM((B, K), jnp.int32),
            pltpu.VMEM((K, DEGW), jnp.float32),
            pltpu.VMEM((K, DEGW), jnp.float32),
            pltpu.SemaphoreType.DMA,
        ],
    )
    return edge, degree


R = 1000
GRID = N // R

_DOT = dict(preferred_element_type=jnp.float32, precision=lax.Precision.HIGHEST)


def _enc_body(x_ref, d0_ref, d1_ref, encW_ref, encb_ref, W1_ref,
              g1_ref, dinv_ref):
    deg = d0_ref[:, 0:1] + d1_ref[:, 0:1] + 1.0
    dinv = lax.rsqrt(deg)
    z = jnp.maximum(jnp.dot(x_ref[...], encW_ref[...], **_DOT) + encb_ref[...], 0.0)
    g1_ref[...] = jnp.dot(z, W1_ref[...], **_DOT) * dinv
    dinv_ref[...] = dinv


def _layer_body(s0_ref, s1_ref, g_ref, dinv_ref, b_ref, Wn_ref, gn_ref):
    dinv = dinv_ref[...]
    h = jax.nn.gelu(dinv * (s0_ref[...] + s1_ref[...] + g_ref[...]) + b_ref[...])
    gn_ref[...] = jnp.dot(h, Wn_ref[...], **_DOT) * dinv


def _final_body(s0_ref, s1_ref, g_ref, dinv_ref, b_ref, decW_ref, decb_ref,
                out_ref):
    dinv = dinv_ref[...]
    h = jax.nn.gelu(dinv * (s0_ref[...] + s1_ref[...] + g_ref[...]) + b_ref[...])
    out_ref[...] = jnp.dot(h, decW_ref[...], **_DOT) + decb_ref[...]


def _rows(shape):
    return pl.BlockSpec((R,) + shape[1:], lambda i: (i,) + (0,) * (len(shape) - 1))


def _whole(shape):
    return pl.BlockSpec(shape, lambda i: (0,) * len(shape))


def _tc_call(body, in_arrays, out_shape):
    in_specs = [_rows(a.shape) if a.shape[0] in (N, NROWS) else _whole(a.shape)
                for a in in_arrays]
    out_specs = jax.tree.map(lambda s: _rows(s.shape), out_shape)
    return pl.pallas_call(
        body,
        grid=(GRID,),
        in_specs=in_specs,
        out_specs=out_specs,
        out_shape=out_shape,
    )(*in_arrays)


def kernel(x, edge_index, enc_W, enc_b, W1, b1, W2, b2, W3, b3, W4, b4,
           dec_W, dec_b):
    ept_real = E // NW
    padt = EPT - ept_real
    src_pad = jnp.zeros((NW, padt), jnp.int32)
    dst_pad = jnp.broadcast_to(
        DUMMY + (jnp.arange(padt, dtype=jnp.int32) % (NROWS - N)), (NW, padt))
    srcR = jnp.concatenate(
        [edge_index[0].reshape(NW, ept_real), src_pad], axis=1).reshape(NW, B, K)
    dstR = jnp.concatenate(
        [edge_index[1].reshape(NW, ept_real), dst_pad], axis=1).reshape(NW, B, K)

    zeros_d = jnp.zeros((K, D), jnp.float32)
    zeros_w = jnp.zeros((K, DEGW), jnp.float32)
    ones_w = jnp.ones((K, DEGW), jnp.float32)

    edge_scatter, degree = _sc_kernels()
    deg_parts = degree(dstR, zeros_w, ones_w)

    f32 = jnp.float32
    g1, dinv = _tc_call(
        _enc_body,
        [x, deg_parts[0], deg_parts[1], enc_W, enc_b.reshape(1, D), W1],
        (jax.ShapeDtypeStruct((N, D), f32), jax.ShapeDtypeStruct((N, 1), f32)),
    )

    g = g1
    for b, Wn in ((b1, W2), (b2, W3), (b3, W4)):
        s_parts = edge_scatter(g, srcR, dstR, zeros_d)
        g = _tc_call(
            _layer_body,
            [s_parts[0], s_parts[1], g, dinv, b.reshape(1, D), Wn],
            jax.ShapeDtypeStruct((N, D), f32),
        )

    s_parts = edge_scatter(g, srcR, dstR, zeros_d)
    out = _tc_call(
        _final_body,
        [s_parts[0], s_parts[1], g, dinv, b4.reshape(1, D), dec_W,
         dec_b.reshape(1, D_OUT)],
        jax.ShapeDtypeStruct((N, D_OUT), f32),
    )
    return out


# speedup vs baseline: 2.3346x; 1.2337x over previous
import functools

import jax
import jax.numpy as jnp
from jax import lax
from jax.experimental import pallas as pl
from jax.experimental.pallas import tpu as pltpu
from jax.experimental.pallas import tpu_sc as plsc

N = 10000
E = 320000
D_IN = 128
D = 64
D_OUT = 4

NC = 2
NS = 16
NW = NC * NS
K = 128
NBUF = 2
B = -(-(-(-E // (NW * K))) // NBUF) * NBUF
EPT = B * K
EP = NW * EPT
NROWS = 10240
DUMMY = N
ZROWS = NROWS // NS
DEGW = 16


def _sc_edge_body(g_hbm, src_hbm, dst_hbm, zeros_hbm, out_hbm,
                  acc, g_s, src_v, dst_v, rows_v, gsem, ssem):
    c = lax.axis_index("c")
    s = lax.axis_index("s")
    wid = c * NS + s
    pltpu.sync_copy(src_hbm.at[wid], src_v)
    pltpu.sync_copy(dst_hbm.at[wid], dst_v)
    last = N - (NS - 1) * ZROWS

    @pl.when(s < NS - 1)
    def _():
        pltpu.sync_copy(g_hbm.at[pl.ds(s * ZROWS, ZROWS)],
                        g_s.at[pl.ds(s * ZROWS, ZROWS)])

    @pl.when(s == NS - 1)
    def _():
        pltpu.sync_copy(g_hbm.at[pl.ds((NS - 1) * ZROWS, last)],
                        g_s.at[pl.ds((NS - 1) * ZROWS, last)])

    for k in range(ZROWS // K):
        pltpu.sync_copy(zeros_hbm, acc.at[pl.ds(s * ZROWS + k * K, K)])
    plsc.subcore_barrier()

    def gather(j, b):
        pltpu.async_copy(g_s.at[src_v.at[j]], rows_v.at[b], gsem.at[b])

    def scatter(j, b):
        pltpu.async_copy(rows_v.at[b], acc.at[dst_v.at[j]], ssem.at[b],
                         add=True)

    def wait_g(b):
        pltpu.make_async_copy(g_s.at[src_v.at[0]], rows_v.at[b],
                              gsem.at[b]).wait()

    def wait_s(b):
        pltpu.make_async_copy(rows_v.at[b], acc.at[dst_v.at[0]],
                              ssem.at[b]).wait()

    for b in range(NBUF):
        gather(b, b)

    @pl.loop(0, B // NBUF - 1)
    def _(i):
        base = i * NBUF
        for b in range(NBUF):
            wait_g(b)
            scatter(base + b, b)
        for b in range(NBUF):
            wait_s(b)
            gather(base + NBUF + b, b)

    tail = B - NBUF
    for b in range(NBUF):
        wait_g(b)
        scatter(tail + b, b)
    for b in range(NBUF):
        wait_s(b)

    plsc.subcore_barrier()
    pltpu.sync_copy(
        acc.at[pl.ds(s * ZROWS, ZROWS)],
        out_hbm.at[c, pl.ds(s * ZROWS, ZROWS)],
    )


_DCHUNK = 8


def _sc_degree_body(dst_hbm, zeros_hbm, ones_hbm, out_hbm,
                    acc, dst_v, ones_v, zbuf_v, sem):
    c = lax.axis_index("c")
    s = lax.axis_index("s")
    wid = c * NS + s
    pltpu.sync_copy(zeros_hbm, zbuf_v)
    pltpu.sync_copy(ones_hbm, ones_v)
    pltpu.sync_copy(dst_hbm.at[wid], dst_v)
    for k in range(ZROWS // K):
        pltpu.sync_copy(zbuf_v, acc.at[pl.ds(s * ZROWS + k * K, K)])
    plsc.subcore_barrier()

    @pl.loop(0, B // _DCHUNK)
    def _(i):
        for k in range(_DCHUNK):
            pltpu.async_copy(ones_v, acc.at[dst_v.at[i * _DCHUNK + k]], sem,
                             add=True)
        for k in range(_DCHUNK):
            pltpu.make_async_copy(ones_v, acc.at[dst_v.at[0]], sem).wait()

    plsc.subcore_barrier()
    pltpu.sync_copy(
        acc.at[pl.ds(s * ZROWS, ZROWS)],
        out_hbm.at[c, pl.ds(s * ZROWS, ZROWS)],
    )


@functools.cache
def _sc_kernels():
    mesh = plsc.VectorSubcoreMesh(
        core_axis_name="c", subcore_axis_name="s", num_cores=NC, num_subcores=NS
    )
    params = pltpu.CompilerParams(use_tc_tiling_on_sc=False)
    edge_params = pltpu.CompilerParams(use_tc_tiling_on_sc=False,
                                       internal_scratch_in_bytes=1 << 20)
    edge = pl.kernel(
        _sc_edge_body,
        compiler_params=edge_params,
        out_type=jax.ShapeDtypeStruct((NC, NROWS, D), jnp.float32),
        mesh=mesh,
        scratch_types=[
            pltpu.VMEM_SHARED((NROWS, D), jnp.float32),
            pltpu.VMEM_SHARED((NROWS, D), jnp.float32),
            pltpu.VMEM((B, K), jnp.int32),
            pltpu.VMEM((B, K), jnp.int32),
            pltpu.VMEM((NBUF, K, D), jnp.float32),
            pltpu.SemaphoreType.DMA((NBUF,)),
            pltpu.SemaphoreType.DMA((NBUF,)),
        ],
    )
    degree = pl.kernel(
        _sc_degree_body,
        compiler_params=params,
        out_type=jax.ShapeDtypeStruct((NC, NROWS, DEGW), jnp.float32),
        mesh=mesh,
        scratch_types=[
            pltpu.VMEM_SHARED((NROWS, DEGW), jnp.float32),
            pltpu.VMEM((B, K), jnp.int32),
            pltpu.VMEM((K, DEGW), jnp.float32),
            pltpu.VMEM((K, DEGW), jnp.float32),
            pltpu.SemaphoreType.DMA,
        ],
    )
    return edge, degree


R = 2000
GRID = N // R

_DOT = dict(preferred_element_type=jnp.float32, precision=lax.Precision.HIGHEST)


RP = R // 2


def _enc_body(x_ref, dp_ref, encW_ref, encb_ref, W1_ref,
              g1_ref, dinv_ref):
    deg = dp_ref[0, :, 0:1] + dp_ref[1, :, 0:1] + 1.0
    dinv = lax.rsqrt(deg)
    z = jnp.maximum(jnp.dot(x_ref[...], encW_ref[...], **_DOT) + encb_ref[...], 0.0)
    g1_ref[...] = jnp.dot(z, W1_ref[...], **_DOT) * dinv
    dinv_ref[...] = dinv


def _packed_hidden(sp_ref, g_ref, dpk_ref, b_ref):
    s = sp_ref[0] + sp_ref[1] + g_ref[...]
    return jax.nn.gelu(dpk_ref[...] * s + b_ref[...])


def _layer_body(sp_ref, g_ref, dpk_ref, b_ref, Wn_ref, gn_ref):
    h = _packed_hidden(sp_ref, g_ref, dpk_ref, b_ref)
    me = jnp.dot(h[:, 0:D], Wn_ref[...], **_DOT)
    mo = jnp.dot(h[:, D:2 * D], Wn_ref[...], **_DOT)
    gn_ref[...] = jnp.concatenate([me, mo], axis=1) * dpk_ref[...]


def _final_body(sp_ref, g_ref, dpk_ref, b_ref, decW_ref, decb_ref, out_ref):
    h = _packed_hidden(sp_ref, g_ref, dpk_ref, b_ref)
    oe = jnp.dot(h[:, 0:D], decW_ref[...], **_DOT) + decb_ref[...]
    oo = jnp.dot(h[:, D:2 * D], decW_ref[...], **_DOT) + decb_ref[...]
    out_ref[...] = jnp.concatenate([oe, oo], axis=1)


def _rows(shape, blk):
    nlead = len(shape) - 2
    if nlead:
        return pl.BlockSpec(shape[:nlead] + (blk, shape[-1]),
                            lambda i: (0,) * nlead + (i, 0))
    return pl.BlockSpec((blk, shape[-1]), lambda i: (i, 0))


def _whole(shape):
    return pl.BlockSpec(shape, lambda i: (0,) * len(shape))


_ROWDIMS = {N: R, NROWS: R, NROWS // 2: RP, N // 2: RP}


def _tc_call(body, in_arrays, out_shape):
    in_specs = [_rows(a.shape, _ROWDIMS[a.shape[-2]])
                if a.shape[-2] in _ROWDIMS else _whole(a.shape)
                for a in in_arrays]
    out_specs = jax.tree.map(lambda s: _rows(s.shape, _ROWDIMS[s.shape[-2]]),
                             out_shape)
    return pl.pallas_call(
        body,
        grid=(GRID,),
        in_specs=in_specs,
        out_specs=out_specs,
        out_shape=out_shape,
    )(*in_arrays)


def kernel(x, edge_index, enc_W, enc_b, W1, b1, W2, b2, W3, b3, W4, b4,
           dec_W, dec_b):
    ept_real = E // NW
    padt = EPT - ept_real
    src_pad = jnp.zeros((NW, padt), jnp.int32)
    dst_pad = jnp.broadcast_to(
        DUMMY + (jnp.arange(padt, dtype=jnp.int32) % (NROWS - N)), (NW, padt))
    srcR = jnp.concatenate(
        [edge_index[0].reshape(NW, ept_real), src_pad], axis=1).reshape(NW, B, K)
    dstR = jnp.concatenate(
        [edge_index[1].reshape(NW, ept_real), dst_pad], axis=1).reshape(NW, B, K)

    zeros_d = jnp.zeros((K, D), jnp.float32)
    zeros_w = jnp.zeros((K, DEGW), jnp.float32)
    ones_w = jnp.ones((K, DEGW), jnp.float32)

    edge_scatter, degree = _sc_kernels()
    deg_parts = degree(dstR, zeros_w, ones_w)

    f32 = jnp.float32
    g1, dinv = _tc_call(
        _enc_body,
        [x, deg_parts, enc_W, enc_b.reshape(1, D), W1],
        (jax.ShapeDtypeStruct((N, D), f32), jax.ShapeDtypeStruct((N, 1), f32)),
    )

    g = g1.reshape(N // 2, 2 * D)
    dinv_pk = jnp.repeat(dinv.reshape(N // 2, 2), D, axis=1)

    for b, Wn in ((b1, W2), (b2, W3), (b3, W4)):
        s_pk = edge_scatter(g.reshape(N, D), srcR, dstR,
                            zeros_d).reshape(NC, NROWS // 2, 2 * D)
        b_pk = jnp.tile(b, 2).reshape(1, 2 * D)
        g = _tc_call(
            _layer_body,
            [s_pk, g, dinv_pk, b_pk, Wn],
            jax.ShapeDtypeStruct((N // 2, 2 * D), f32),
        )

    s_pk = edge_scatter(g.reshape(N, D), srcR, dstR,
                        zeros_d).reshape(NC, NROWS // 2, 2 * D)
    out_pk = _tc_call(
        _final_body,
        [s_pk, g, dinv_pk, jnp.tile(b4, 2).reshape(1, 2 * D), dec_W,
         dec_b.reshape(1, D_OUT)],
        jax.ShapeDtypeStruct((N // 2, 2 * D_OUT), f32),
    )
    return out_pk.reshape(N, D_OUT)
